# Initial kernel scaffold; baseline (speedup 1.0000x reference)
import jax
import jax.numpy as jnp
from jax.experimental import pallas as pl


def kernel(fmap1, fmap2, coords):
    raise NotImplementedError("write your pallas kernel here")


# speedup vs baseline: 5.1861x; 5.1861x over previous
import functools

import jax
import jax.numpy as jnp
from jax import lax
from jax.experimental import pallas as pl
from jax.experimental.pallas import tpu as pltpu
from jax.experimental.pallas import tpu_sc as plsc

NQ = 4096
CCH = 256
W0 = 64
NLVL = 4
RAD = 4
NT = 9
PYR_COLS = 4096 + 1024 + 256 + 64
ROWS_PER_Q = PYR_COLS // 64
OUTC = NLVL * NT * NT
OUTP = 336
QB = 128
NTILES = 32
QPT = NQ // NTILES


def _pool_mat(n):
    m = n // 2
    a = (jnp.arange(n) // 2)
    ay = (a[:, None] == jnp.arange(m)[None, :]).astype(jnp.float32)
    g = jnp.einsum('yY,xX->yxYX', ay, ay).reshape(n * n, m * m) * 0.25
    return g


def _tc_pyramid(f1t, f2m, g1, g2, g3):
    def body(f1_ref, f2_ref, g1_ref, g2_ref, g3_ref, out_ref, f2all_ref):
        @pl.when(pl.program_id(0) == 0)
        def _():
            f2 = f2_ref[...]
            nrm = jnp.sqrt(jnp.sum(f2 * f2, axis=0, keepdims=True)) + 1e-8
            f2n = f2 / nrm
            f2all_ref[:, 0:4096] = f2n
            p1 = jnp.dot(f2n, g1_ref[...],
                         preferred_element_type=jnp.float32,
                         precision=lax.Precision.HIGHEST)
            f2all_ref[:, 4096:5120] = p1
            p2 = jnp.dot(p1, g2_ref[...],
                         preferred_element_type=jnp.float32,
                         precision=lax.Precision.HIGHEST)
            f2all_ref[:, 5120:5376] = p2
            f2all_ref[:, 5376:5440] = jnp.dot(
                p2, g3_ref[...], preferred_element_type=jnp.float32,
                precision=lax.Precision.HIGHEST)

        f1 = f1_ref[...]
        nrm1 = jnp.sqrt(jnp.sum(f1 * f1, axis=1, keepdims=True)) + 1e-8
        f1n = f1 / nrm1
        out_ref[...] = jnp.dot(f1n, f2all_ref[...],
                               preferred_element_type=jnp.float32,
                               precision=lax.Precision.HIGHEST)

    return pl.pallas_call(
        body,
        grid=(NQ // QB,),
        in_specs=[
            pl.BlockSpec((QB, CCH), lambda i: (i, 0)),
            pl.BlockSpec((CCH, 4096), lambda i: (0, 0)),
            pl.BlockSpec((4096, 1024), lambda i: (0, 0)),
            pl.BlockSpec((1024, 256), lambda i: (0, 0)),
            pl.BlockSpec((256, 64), lambda i: (0, 0)),
        ],
        out_specs=pl.BlockSpec((QB, PYR_COLS), lambda i: (i, 0)),
        out_shape=jax.ShapeDtypeStruct((NQ, PYR_COLS), jnp.float32),
        scratch_shapes=[pltpu.VMEM((CCH, PYR_COLS), jnp.float32)],
    )(f1t, f2m, g1, g2, g3)


_LEVELS = ((64, 1.0, 0), (32, 0.5, 81), (16, 0.25, 162), (8, 0.125, 243))


def _sc_sample(pyr_flat, cq):
    mesh = plsc.VectorSubcoreMesh(core_axis_name="c", subcore_axis_name="s")

    @functools.partial(
        pl.kernel,
        out_type=jax.ShapeDtypeStruct((NQ * OUTP,), jnp.float32),
        mesh=mesh,
        compiler_params=pltpu.CompilerParams(needs_layout_passes=False),
        scratch_types=[
            pltpu.VMEM((2 * QPT + 16,), jnp.float32),
            pltpu.VMEM((640,), jnp.float32),
            pltpu.VMEM((384,), jnp.float32),
            pltpu.VMEM((320,), jnp.float32),
            pltpu.VMEM((OUTP,), jnp.float32),
            pltpu.SemaphoreType.DMA,
            pltpu.SemaphoreType.DMA,
            pltpu.SemaphoreType.DMA,
            pltpu.SemaphoreType.DMA,
        ],
    )
    def ker(pyr_hbm, cq_hbm, out_hbm, cbuf, b0, b1, b23, obuf,
            sem, sem1, sem2, osem):
        wid = lax.axis_index("s") * 2 + lax.axis_index("c")
        q0 = wid * QPT
        pltpu.sync_copy(cq_hbm.at[pl.ds(q0 * 2, 2 * QPT)],
                        cbuf.at[pl.ds(0, 2 * QPT)])
        def floori(s):
            r = s.astype(jnp.int32)
            return jnp.where(r.astype(jnp.float32) > s, r - 1, r)

        def body(qi, carry):
            iota = lax.iota(jnp.int32, 16)
            lane9 = iota < NT
            q = q0 + qi
            cv = cbuf[pl.ds(2 * qi, 16)]
            c0 = cv[0]
            c1 = cv[1]
            base = q * ROWS_PER_Q
            y00 = floori(c1) - 4
            ts0 = jnp.clip(y00, 0, 54)
            y01 = floori(c1 * 0.5) - 4
            ts1 = jnp.clip(y01 // 2, 0, 10)
            cp0 = pltpu.make_async_copy(
                pyr_hbm.at[pl.ds((base + ts0) * 64, 640)], b0, sem)
            cp1 = pltpu.make_async_copy(
                pyr_hbm.at[pl.ds((base + 64 + ts1) * 64, 384)], b1, sem1)
            cp2 = pltpu.make_async_copy(
                pyr_hbm.at[pl.ds((base + 80) * 64, 320)], b23, sem2)
            cp0.start()
            cp1.start()
            cp2.start()
            cp0.wait()
            cp1.wait()
            cp2.wait()

            def bcast_i(s):
                return jnp.broadcast_to(s, (16,)).astype(jnp.int32)

            def bcast_f(s):
                return jnp.broadcast_to(s, (16,)).astype(jnp.float32)

            def do_level(dref, dbase, ybase, w, inv_s, obase):
                xs = c0 * inv_s
                ys = c1 * inv_s
                xi = floori(xs)
                fx = xs - xi.astype(jnp.float32)
                yi = floori(ys)
                fy = ys - yi.astype(jnp.float32)
                x0 = xi - 4
                y0 = yi - 4
                one_v = jnp.full((16,), 1.0, jnp.float32)
                zero_v = jnp.zeros((16,), jnp.float32)
                xa = bcast_i(x0) + iota
                xb = xa + 1
                va = jnp.where((xa >= 0) & (xa <= w - 1), one_v, zero_v)
                vb = jnp.where((xb >= 0) & (xb <= w - 1), one_v, zero_v)
                ia = jnp.clip(xa, 0, w - 1)
                ib = jnp.clip(xb, 0, w - 1)
                gav = bcast_f(1.0 - fx) * va
                gbv = bcast_f(fx) * vb
                hrows = []
                for jj in range(10):
                    l = jnp.clip(y0 + jj, 0, w - 1) - ybase
                    offv = bcast_i(dbase + l * w)
                    ra = plsc.load_gather(dref, [offv + ia])
                    rb = plsc.load_gather(dref, [offv + ib])
                    hrows.append(gav * ra + gbv * rb)
                oidx = obase + iota * NT
                for k in range(NT):
                    ya = y0 + k
                    yb = ya + 1
                    wa = jnp.where((ya >= 0) & (ya <= w - 1), 1.0 - fy, 0.0)
                    wb = jnp.where((yb >= 0) & (yb <= w - 1), fy, 0.0)
                    outk = bcast_f(wa) * hrows[k] + bcast_f(wb) * hrows[k + 1]
                    plsc.store_scatter(obuf, [oidx + k], outk, mask=lane9)

            do_level(b0, 0, ts0, 64, 1.0, 0)
            do_level(b1, 0, 2 * ts1, 32, 0.5, 81)
            do_level(b23, 0, 0, 16, 0.25, 162)
            do_level(b23, 256, 0, 8, 0.125, 243)

            ocp = pltpu.make_async_copy(
                obuf, out_hbm.at[pl.ds(q * OUTP, OUTP)], osem)
            ocp.start()
            ocp.wait()
            return carry

        lax.fori_loop(0, QPT, body, 0)

    return ker(pyr_flat, cq)


def kernel(fmap1, fmap2, coords):
    B, C, H, W = fmap1.shape
    f1t = fmap1.reshape(C, H * W).T
    f2m = fmap2.reshape(C, H * W)
    g1 = _pool_mat(64)
    g2 = _pool_mat(32)
    g3 = _pool_mat(16)
    pyr = _tc_pyramid(f1t, f2m, g1, g2, g3)
    cq = coords.reshape(2, H * W).T.reshape(-1)
    out = _sc_sample(pyr.reshape(-1), cq)
    out = out.reshape(NQ, OUTP)[:, :OUTC]
    return out.reshape(1, OUTC, H, W).astype(jnp.float32)


# speedup vs baseline: 5.8949x; 1.1367x over previous
import functools

import jax
import jax.numpy as jnp
from jax import lax
from jax.experimental import pallas as pl
from jax.experimental.pallas import tpu as pltpu
from jax.experimental.pallas import tpu_sc as plsc

NQ = 4096
CCH = 256
W0 = 64
NLVL = 4
RAD = 4
NT = 9
PYR_COLS = 4096 + 1024 + 256 + 64
ROWS_PER_Q = PYR_COLS // 64
OUTC = NLVL * NT * NT
OUTP = 336
QB = 128
NTILES = 32
QPT = NQ // NTILES


def _pool_mat(n):
    m = n // 2
    a = (jnp.arange(n) // 2)
    ay = (a[:, None] == jnp.arange(m)[None, :]).astype(jnp.float32)
    g = jnp.einsum('yY,xX->yxYX', ay, ay).reshape(n * n, m * m) * 0.25
    return g


def _tc_pyramid(f1t, f2m, g1, g2, g3):
    def body(f1_ref, f2_ref, g1_ref, g2_ref, g3_ref, out_ref, f2all_ref):
        @pl.when(pl.program_id(0) == 0)
        def _():
            f2 = f2_ref[...]
            nrm = jnp.sqrt(jnp.sum(f2 * f2, axis=0, keepdims=True)) + 1e-8
            f2n = f2 / nrm
            f2all_ref[:, 0:4096] = f2n
            p1 = jnp.dot(f2n, g1_ref[...],
                         preferred_element_type=jnp.float32,
                         precision=lax.Precision.HIGHEST)
            f2all_ref[:, 4096:5120] = p1
            p2 = jnp.dot(p1, g2_ref[...],
                         preferred_element_type=jnp.float32,
                         precision=lax.Precision.HIGHEST)
            f2all_ref[:, 5120:5376] = p2
            f2all_ref[:, 5376:5440] = jnp.dot(
                p2, g3_ref[...], preferred_element_type=jnp.float32,
                precision=lax.Precision.HIGHEST)

        f1 = f1_ref[...]
        nrm1 = jnp.sqrt(jnp.sum(f1 * f1, axis=1, keepdims=True)) + 1e-8
        f1n = f1 / nrm1
        out_ref[...] = jnp.dot(f1n, f2all_ref[...],
                               preferred_element_type=jnp.float32,
                               precision=lax.Precision.DEFAULT)

    return pl.pallas_call(
        body,
        grid=(NQ // QB,),
        in_specs=[
            pl.BlockSpec((QB, CCH), lambda i: (i, 0)),
            pl.BlockSpec((CCH, 4096), lambda i: (0, 0)),
            pl.BlockSpec((4096, 1024), lambda i: (0, 0)),
            pl.BlockSpec((1024, 256), lambda i: (0, 0)),
            pl.BlockSpec((256, 64), lambda i: (0, 0)),
        ],
        out_specs=pl.BlockSpec((QB, PYR_COLS), lambda i: (i, 0)),
        out_shape=jax.ShapeDtypeStruct((NQ, PYR_COLS), jnp.float32),
        scratch_shapes=[pltpu.VMEM((CCH, PYR_COLS), jnp.float32)],
    )(f1t, f2m, g1, g2, g3)


_LEVELS = ((64, 1.0, 0), (32, 0.5, 81), (16, 0.25, 162), (8, 0.125, 243))


def _sc_sample(pyr_flat, cq):
    mesh = plsc.VectorSubcoreMesh(core_axis_name="c", subcore_axis_name="s")

    @functools.partial(
        pl.kernel,
        out_type=jax.ShapeDtypeStruct((NQ * OUTP,), jnp.float32),
        mesh=mesh,
        compiler_params=pltpu.CompilerParams(needs_layout_passes=False),
        scratch_types=[
            pltpu.VMEM((2 * QPT + 16,), jnp.float32),
            pltpu.VMEM((640,), jnp.float32),
            pltpu.VMEM((384,), jnp.float32),
            pltpu.VMEM((320,), jnp.float32),
            pltpu.VMEM((OUTP,), jnp.float32),
            pltpu.SemaphoreType.DMA,
            pltpu.SemaphoreType.DMA,
            pltpu.SemaphoreType.DMA,
            pltpu.SemaphoreType.DMA,
        ],
    )
    def ker(pyr_hbm, cq_hbm, out_hbm, cbuf, b0, b1, b23, obuf,
            sem, sem1, sem2, osem):
        wid = lax.axis_index("s") * 2 + lax.axis_index("c")
        q0 = wid * QPT
        pltpu.sync_copy(cq_hbm.at[pl.ds(q0 * 2, 2 * QPT)],
                        cbuf.at[pl.ds(0, 2 * QPT)])
        def floori(s):
            r = s.astype(jnp.int32)
            return jnp.where(r.astype(jnp.float32) > s, r - 1, r)

        def body(qi, carry):
            iota = lax.iota(jnp.int32, 16)
            lane9 = iota < NT
            q = q0 + qi
            cv = cbuf[pl.ds(2 * qi, 16)]
            c0 = cv[0]
            c1 = cv[1]
            base = q * ROWS_PER_Q
            y00 = floori(c1) - 4
            ts0 = jnp.clip(y00, 0, 54)
            y01 = floori(c1 * 0.5) - 4
            ts1 = jnp.clip(y01 // 2, 0, 10)
            cp0 = pltpu.make_async_copy(
                pyr_hbm.at[pl.ds((base + ts0) * 64, 640)], b0, sem)
            cp1 = pltpu.make_async_copy(
                pyr_hbm.at[pl.ds((base + 64 + ts1) * 64, 384)], b1, sem1)
            cp2 = pltpu.make_async_copy(
                pyr_hbm.at[pl.ds((base + 80) * 64, 320)], b23, sem2)
            cp0.start()
            cp1.start()
            cp2.start()
            cp0.wait()
            cp1.wait()
            cp2.wait()

            def bcast_i(s):
                return jnp.broadcast_to(s, (16,)).astype(jnp.int32)

            def bcast_f(s):
                return jnp.broadcast_to(s, (16,)).astype(jnp.float32)

            def do_level(dref, dbase, ybase, w, inv_s, obase):
                xs = c0 * inv_s
                ys = c1 * inv_s
                xi = floori(xs)
                fx = xs - xi.astype(jnp.float32)
                yi = floori(ys)
                fy = ys - yi.astype(jnp.float32)
                x0 = xi - 4
                y0 = yi - 4
                one_v = jnp.full((16,), 1.0, jnp.float32)
                zero_v = jnp.zeros((16,), jnp.float32)
                xa = bcast_i(x0) + iota
                xb = xa + 1
                va = jnp.where((xa >= 0) & (xa <= w - 1), one_v, zero_v)
                vb = jnp.where((xb >= 0) & (xb <= w - 1), one_v, zero_v)
                ia = jnp.clip(xa, 0, w - 1)
                ib = jnp.clip(xb, 0, w - 1)
                gav = bcast_f(1.0 - fx) * va
                gbv = bcast_f(fx) * vb
                hrows = []
                for jj in range(10):
                    l = jnp.clip(y0 + jj, 0, w - 1) - ybase
                    offv = bcast_i(dbase + l * w)
                    ra = plsc.load_gather(dref, [offv + ia])
                    rb = plsc.load_gather(dref, [offv + ib])
                    hrows.append(gav * ra + gbv * rb)
                oidx = obase + iota * NT
                for k in range(NT):
                    ya = y0 + k
                    yb = ya + 1
                    wa = jnp.where((ya >= 0) & (ya <= w - 1), 1.0 - fy, 0.0)
                    wb = jnp.where((yb >= 0) & (yb <= w - 1), fy, 0.0)
                    outk = bcast_f(wa) * hrows[k] + bcast_f(wb) * hrows[k + 1]
                    plsc.store_scatter(obuf, [oidx + k], outk, mask=lane9)

            do_level(b0, 0, ts0, 64, 1.0, 0)
            do_level(b1, 0, 2 * ts1, 32, 0.5, 81)
            do_level(b23, 0, 0, 16, 0.25, 162)
            do_level(b23, 256, 0, 8, 0.125, 243)

            ocp = pltpu.make_async_copy(
                obuf, out_hbm.at[pl.ds(q * OUTP, OUTP)], osem)
            ocp.start()
            ocp.wait()
            return carry

        lax.fori_loop(0, QPT, body, 0)

    return ker(pyr_flat, cq)


def kernel(fmap1, fmap2, coords):
    B, C, H, W = fmap1.shape
    f1t = fmap1.reshape(C, H * W).T
    f2m = fmap2.reshape(C, H * W)
    g1 = _pool_mat(64)
    g2 = _pool_mat(32)
    g3 = _pool_mat(16)
    pyr = _tc_pyramid(f1t, f2m, g1, g2, g3)
    cq = coords.reshape(2, H * W).T.reshape(-1)
    out = _sc_sample(pyr.reshape(-1), cq)
    out = out.reshape(NQ, OUTP)[:, :OUTC]
    return out.reshape(1, OUTC, H, W).astype(jnp.float32)


# speedup vs baseline: 7.0751x; 1.2002x over previous
import functools

import jax
import jax.numpy as jnp
from jax import lax
from jax.experimental import pallas as pl
from jax.experimental.pallas import tpu as pltpu
from jax.experimental.pallas import tpu_sc as plsc

NQ = 4096
CCH = 256
W0 = 64
NLVL = 4
RAD = 4
NT = 9
PYR_COLS = 4096 + 1024 + 256 + 64
ROWS_PER_Q = PYR_COLS // 64
OUTC = NLVL * NT * NT
OUTP = 336
QB = 128
NTILES = 32
QPT = NQ // NTILES


def _pool_mat(n):
    m = n // 2
    a = (jnp.arange(n) // 2)
    ay = (a[:, None] == jnp.arange(m)[None, :]).astype(jnp.float32)
    g = jnp.einsum('yY,xX->yxYX', ay, ay).reshape(n * n, m * m) * 0.25
    return g


def _tc_pyramid(f1t, f2m, g1, g2, g3):
    def body(f1_ref, f2_ref, g1_ref, g2_ref, g3_ref, out_ref, f2all_ref):
        @pl.when(pl.program_id(0) == 0)
        def _():
            f2 = f2_ref[...]
            nrm = jnp.sqrt(jnp.sum(f2 * f2, axis=0, keepdims=True)) + 1e-8
            f2n = f2 / nrm
            f2all_ref[:, 0:4096] = f2n
            p1 = jnp.dot(f2n, g1_ref[...],
                         preferred_element_type=jnp.float32,
                         precision=lax.Precision.DEFAULT)
            f2all_ref[:, 4096:5120] = p1
            p2 = jnp.dot(p1, g2_ref[...],
                         preferred_element_type=jnp.float32,
                         precision=lax.Precision.DEFAULT)
            f2all_ref[:, 5120:5376] = p2
            f2all_ref[:, 5376:5440] = jnp.dot(
                p2, g3_ref[...], preferred_element_type=jnp.float32,
                precision=lax.Precision.DEFAULT)

        f1 = f1_ref[...]
        nrm1 = jnp.sqrt(jnp.sum(f1 * f1, axis=1, keepdims=True)) + 1e-8
        f1n = f1 / nrm1
        out_ref[...] = jnp.dot(f1n, f2all_ref[...],
                               preferred_element_type=jnp.float32,
                               precision=lax.Precision.DEFAULT)

    return pl.pallas_call(
        body,
        grid=(NQ // QB,),
        in_specs=[
            pl.BlockSpec((QB, CCH), lambda i: (i, 0)),
            pl.BlockSpec((CCH, 4096), lambda i: (0, 0)),
            pl.BlockSpec((4096, 1024), lambda i: (0, 0)),
            pl.BlockSpec((1024, 256), lambda i: (0, 0)),
            pl.BlockSpec((256, 64), lambda i: (0, 0)),
        ],
        out_specs=pl.BlockSpec((QB, PYR_COLS), lambda i: (i, 0)),
        out_shape=jax.ShapeDtypeStruct((NQ, PYR_COLS), jnp.float32),
        scratch_shapes=[pltpu.VMEM((CCH, PYR_COLS), jnp.float32)],
    )(f1t, f2m, g1, g2, g3)


_LEVELS = ((64, 1.0, 0), (32, 0.5, 81), (16, 0.25, 162), (8, 0.125, 243))


def _sc_sample(pyr_flat, cq):
    mesh = plsc.VectorSubcoreMesh(core_axis_name="c", subcore_axis_name="s")

    @functools.partial(
        pl.kernel,
        out_type=jax.ShapeDtypeStruct((NQ * OUTP,), jnp.float32),
        mesh=mesh,
        compiler_params=pltpu.CompilerParams(needs_layout_passes=False),
        scratch_types=[
            pltpu.VMEM((2 * QPT + 16,), jnp.float32),
            pltpu.VMEM((2 * 640,), jnp.float32),
            pltpu.VMEM((2 * 384,), jnp.float32),
            pltpu.VMEM((2 * 320,), jnp.float32),
            pltpu.VMEM((2 * OUTP,), jnp.float32),
            pltpu.SemaphoreType.DMA,
            pltpu.SemaphoreType.DMA,
            pltpu.SemaphoreType.DMA,
            pltpu.SemaphoreType.DMA,
            pltpu.SemaphoreType.DMA,
            pltpu.SemaphoreType.DMA,
            pltpu.SemaphoreType.DMA,
        ],
    )
    def ker(pyr_hbm, cq_hbm, out_hbm, cbuf, b0, b1, b23, obuf,
            sa0, sa1, sa2, sb0, sb1, sb2, osem):
        wid = lax.axis_index("s") * 2 + lax.axis_index("c")
        q0 = wid * QPT
        pltpu.sync_copy(cq_hbm.at[pl.ds(q0 * 2, 2 * QPT)],
                        cbuf.at[pl.ds(0, 2 * QPT)])

        def floori(s):
            r = s.astype(jnp.int32)
            return jnp.where(r.astype(jnp.float32) > s, r - 1, r)

        def windows(qi):
            cv = cbuf[pl.ds(2 * qi, 16)]
            c0 = cv[0]
            c1 = cv[1]
            ts0 = jnp.clip(floori(c1) - 4, 0, 54)
            ts1 = jnp.clip((floori(c1 * 0.5) - 4) // 2, 0, 10)
            return c0, c1, ts0, ts1

        def start_in(qi, h, s0, s1, s2):
            _, _, ts0, ts1 = windows(qi)
            base = (q0 + qi) * ROWS_PER_Q
            pltpu.make_async_copy(
                pyr_hbm.at[pl.ds((base + ts0) * 64, 640)],
                b0.at[pl.ds(h * 640, 640)], s0).start()
            pltpu.make_async_copy(
                pyr_hbm.at[pl.ds((base + 64 + ts1) * 64, 384)],
                b1.at[pl.ds(h * 384, 384)], s1).start()
            pltpu.make_async_copy(
                pyr_hbm.at[pl.ds((base + 80) * 64, 320)],
                b23.at[pl.ds(h * 320, 320)], s2).start()

        def wait_in(h, s0, s1, s2):
            pltpu.make_async_copy(pyr_hbm.at[pl.ds(0, 640)],
                                  b0.at[pl.ds(h * 640, 640)], s0).wait()
            pltpu.make_async_copy(pyr_hbm.at[pl.ds(0, 384)],
                                  b1.at[pl.ds(h * 384, 384)], s1).wait()
            pltpu.make_async_copy(pyr_hbm.at[pl.ds(0, 320)],
                                  b23.at[pl.ds(h * 320, 320)], s2).wait()

        def wait_out():
            pltpu.make_async_copy(obuf.at[pl.ds(0, OUTP)],
                                  out_hbm.at[pl.ds(0, OUTP)], osem).wait()

        start_in(0, 0, sa0, sa1, sa2)
        pltpu.make_async_copy(obuf.at[pl.ds(0, OUTP)],
                              out_hbm.at[pl.ds(q0 * OUTP, OUTP)],
                              osem).start()

        def body(qi, carry):
            iota = lax.iota(jnp.int32, 16)
            lane9 = iota < NT
            phase = qi & 1
            nqi = jnp.where(qi + 1 < QPT, qi + 1, 0)

            @pl.when(phase == 0)
            def _():
                start_in(nqi, 1, sb0, sb1, sb2)
                wait_in(0, sa0, sa1, sa2)

            @pl.when(phase == 1)
            def _():
                start_in(nqi, 0, sa0, sa1, sa2)
                wait_in(1, sb0, sb1, sb2)

            c0, c1, ts0, ts1 = windows(qi)
            o0 = phase * 640
            o1 = phase * 384
            o23 = phase * 320
            oo = phase * OUTP

            def bcast_i(s):
                return jnp.broadcast_to(s, (16,)).astype(jnp.int32)

            def bcast_f(s):
                return jnp.broadcast_to(s, (16,)).astype(jnp.float32)

            def do_level(dref, dbase, ybase, w, inv_s, obase):
                xs = c0 * inv_s
                ys = c1 * inv_s
                xi = floori(xs)
                fx = xs - xi.astype(jnp.float32)
                yi = floori(ys)
                fy = ys - yi.astype(jnp.float32)
                x0 = xi - 4
                y0 = yi - 4
                one_v = jnp.full((16,), 1.0, jnp.float32)
                zero_v = jnp.zeros((16,), jnp.float32)
                xa = bcast_i(x0) + iota
                xb = xa + 1
                va = jnp.where((xa >= 0) & (xa <= w - 1), one_v, zero_v)
                vb = jnp.where((xb >= 0) & (xb <= w - 1), one_v, zero_v)
                ia = jnp.clip(xa, 0, w - 1)
                ib = jnp.clip(xb, 0, w - 1)
                gav = bcast_f(1.0 - fx) * va
                gbv = bcast_f(fx) * vb
                hrows = []
                for jj in range(10):
                    l = jnp.clip(y0 + jj, 0, w - 1) - ybase
                    offv = bcast_i(dbase + l * w)
                    ra = plsc.load_gather(dref, [offv + ia])
                    rb = plsc.load_gather(dref, [offv + ib])
                    hrows.append(gav * ra + gbv * rb)
                oidx = oo + obase + iota * NT
                for k in range(NT):
                    ya = y0 + k
                    yb = ya + 1
                    wa = jnp.where((ya >= 0) & (ya <= w - 1), 1.0 - fy, 0.0)
                    wb = jnp.where((yb >= 0) & (yb <= w - 1), fy, 0.0)
                    outk = bcast_f(wa) * hrows[k] + bcast_f(wb) * hrows[k + 1]
                    plsc.store_scatter(obuf, [oidx + k], outk, mask=lane9)

            do_level(b0, o0, ts0, 64, 1.0, 0)
            do_level(b1, o1, 2 * ts1, 32, 0.5, 81)
            do_level(b23, o23, 0, 16, 0.25, 162)
            do_level(b23, o23 + 256, 0, 8, 0.125, 243)

            wait_out()
            pltpu.make_async_copy(
                obuf.at[pl.ds(oo, OUTP)],
                out_hbm.at[pl.ds((q0 + qi) * OUTP, OUTP)], osem).start()
            return carry

        lax.fori_loop(0, QPT, body, 0)
        wait_out()
        wait_in(0, sa0, sa1, sa2)

    return ker(pyr_flat, cq)


def kernel(fmap1, fmap2, coords):
    B, C, H, W = fmap1.shape
    f1t = fmap1.reshape(C, H * W).T
    f2m = fmap2.reshape(C, H * W)
    g1 = _pool_mat(64)
    g2 = _pool_mat(32)
    g3 = _pool_mat(16)
    pyr = _tc_pyramid(f1t, f2m, g1, g2, g3)
    cq = coords.reshape(2, H * W).T.reshape(-1)
    out = _sc_sample(pyr.reshape(-1), cq)
    out = out.reshape(NQ, OUTP)[:, :OUTC]
    return out.reshape(1, OUTC, H, W).astype(jnp.float32)


# speedup vs baseline: 8.9795x; 1.2692x over previous
import functools

import jax
import jax.numpy as jnp
from jax import lax
from jax.experimental import pallas as pl
from jax.experimental.pallas import tpu as pltpu
from jax.experimental.pallas import tpu_sc as plsc

NQ = 4096
CCH = 256
W0 = 64
NLVL = 4
RAD = 4
NT = 9
PYR_COLS = 4096 + 1024 + 256 + 64
ROWS_PER_Q = PYR_COLS // 64
OUTC = NLVL * NT * NT
OUTP = 336
QB = 128
NTILES = 32
QPT = NQ // NTILES


def _pool_mat(n):
    m = n // 2
    a = (jnp.arange(n) // 2)
    ay = (a[:, None] == jnp.arange(m)[None, :]).astype(jnp.float32)
    g = jnp.einsum('yY,xX->yxYX', ay, ay).reshape(n * n, m * m) * 0.25
    return g


def _tc_pyramid(f1t, f2m, g1, g2, g3):
    def body(f1_ref, f2_ref, g1_ref, g2_ref, g3_ref, out_ref, f2all_ref):
        @pl.when(pl.program_id(0) == 0)
        def _():
            f2 = f2_ref[...]
            nrm = jnp.sqrt(jnp.sum(f2 * f2, axis=0, keepdims=True)) + 1e-8
            f2n = f2 / nrm
            f2all_ref[:, 0:4096] = f2n
            p1 = jnp.dot(f2n, g1_ref[...],
                         preferred_element_type=jnp.float32,
                         precision=lax.Precision.DEFAULT)
            f2all_ref[:, 4096:5120] = p1
            p2 = jnp.dot(p1, g2_ref[...],
                         preferred_element_type=jnp.float32,
                         precision=lax.Precision.DEFAULT)
            f2all_ref[:, 5120:5376] = p2
            f2all_ref[:, 5376:5440] = jnp.dot(
                p2, g3_ref[...], preferred_element_type=jnp.float32,
                precision=lax.Precision.DEFAULT)

        f1 = f1_ref[...]
        nrm1 = jnp.sqrt(jnp.sum(f1 * f1, axis=1, keepdims=True)) + 1e-8
        f1n = f1 / nrm1
        corr = jnp.dot(f1n, f2all_ref[...],
                       preferred_element_type=jnp.float32,
                       precision=lax.Precision.DEFAULT)
        for cb in range(42):
            out_ref[:, cb, :] = corr[:, cb * 128:(cb + 1) * 128]
        out_ref[:, 42, 0:64] = corr[:, 5376:5440]

    return pl.pallas_call(
        body,
        grid=(NQ // QB,),
        in_specs=[
            pl.BlockSpec((QB, CCH), lambda i: (i, 0)),
            pl.BlockSpec((CCH, 4096), lambda i: (0, 0)),
            pl.BlockSpec((4096, 1024), lambda i: (0, 0)),
            pl.BlockSpec((1024, 256), lambda i: (0, 0)),
            pl.BlockSpec((256, 64), lambda i: (0, 0)),
        ],
        out_specs=pl.BlockSpec((QB, 48, 128), lambda i: (i, 0, 0)),
        out_shape=jax.ShapeDtypeStruct((NQ, 48, 128), jnp.float32),
        scratch_shapes=[pltpu.VMEM((CCH, PYR_COLS), jnp.float32)],
    )(f1t, f2m, g1, g2, g3)


_LEVELS = ((64, 1.0, 0), (32, 0.5, 81), (16, 0.25, 162), (8, 0.125, 243))


def _sc_sample(pyr_flat, cq):
    mesh = plsc.VectorSubcoreMesh(core_axis_name="c", subcore_axis_name="s")

    @functools.partial(
        pl.kernel,
        out_type=jax.ShapeDtypeStruct((NQ * OUTP,), jnp.float32),
        mesh=mesh,
        compiler_params=pltpu.CompilerParams(needs_layout_passes=False),
        scratch_types=[
            pltpu.VMEM((2 * QPT + 16,), jnp.float32),
            pltpu.VMEM((2 * 640,), jnp.float32),
            pltpu.VMEM((2 * 384,), jnp.float32),
            pltpu.VMEM((2 * 320,), jnp.float32),
            pltpu.VMEM((2 * OUTP,), jnp.float32),
            pltpu.SemaphoreType.DMA,
            pltpu.SemaphoreType.DMA,
            pltpu.SemaphoreType.DMA,
            pltpu.SemaphoreType.DMA,
            pltpu.SemaphoreType.DMA,
            pltpu.SemaphoreType.DMA,
            pltpu.SemaphoreType.DMA,
        ],
    )
    def ker(pyr_hbm, cq_hbm, out_hbm, cbuf, b0, b1, b23, obuf,
            sa0, sa1, sa2, sb0, sb1, sb2, osem):
        wid = lax.axis_index("s") * 2 + lax.axis_index("c")
        q0 = wid * QPT
        pltpu.sync_copy(cq_hbm.at[pl.ds(q0 * 2, 2 * QPT)],
                        cbuf.at[pl.ds(0, 2 * QPT)])

        def floori(s):
            r = s.astype(jnp.int32)
            return jnp.where(r.astype(jnp.float32) > s, r - 1, r)

        def windows(qi):
            cv = cbuf[pl.ds(2 * qi, 16)]
            c0 = cv[0]
            c1 = cv[1]
            ts0 = jnp.clip(floori(c1) - 4, 0, 54)
            ts1 = jnp.clip((floori(c1 * 0.5) - 4) // 2, 0, 10)
            return c0, c1, ts0, ts1

        def start_in(qi, h, s0, s1, s2):
            _, _, ts0, ts1 = windows(qi)
            base = (q0 + qi) * 96
            pltpu.make_async_copy(
                pyr_hbm.at[pl.ds((base + ts0) * 64, 640)],
                b0.at[pl.ds(h * 640, 640)], s0).start()
            pltpu.make_async_copy(
                pyr_hbm.at[pl.ds((base + 64 + ts1) * 64, 384)],
                b1.at[pl.ds(h * 384, 384)], s1).start()
            pltpu.make_async_copy(
                pyr_hbm.at[pl.ds((base + 80) * 64, 320)],
                b23.at[pl.ds(h * 320, 320)], s2).start()

        def wait_in(h, s0, s1, s2):
            pltpu.make_async_copy(pyr_hbm.at[pl.ds(0, 640)],
                                  b0.at[pl.ds(h * 640, 640)], s0).wait()
            pltpu.make_async_copy(pyr_hbm.at[pl.ds(0, 384)],
                                  b1.at[pl.ds(h * 384, 384)], s1).wait()
            pltpu.make_async_copy(pyr_hbm.at[pl.ds(0, 320)],
                                  b23.at[pl.ds(h * 320, 320)], s2).wait()

        def wait_out():
            pltpu.make_async_copy(obuf.at[pl.ds(0, OUTP)],
                                  out_hbm.at[pl.ds(0, OUTP)], osem).wait()

        start_in(0, 0, sa0, sa1, sa2)
        pltpu.make_async_copy(obuf.at[pl.ds(0, OUTP)],
                              out_hbm.at[pl.ds(q0 * OUTP, OUTP)],
                              osem).start()

        def body(qi, carry):
            iota = lax.iota(jnp.int32, 16)
            lane9 = iota < NT
            phase = qi & 1
            nqi = jnp.where(qi + 1 < QPT, qi + 1, 0)

            @pl.when(phase == 0)
            def _():
                start_in(nqi, 1, sb0, sb1, sb2)
                wait_in(0, sa0, sa1, sa2)

            @pl.when(phase == 1)
            def _():
                start_in(nqi, 0, sa0, sa1, sa2)
                wait_in(1, sb0, sb1, sb2)

            c0, c1, ts0, ts1 = windows(qi)
            o0 = phase * 640
            o1 = phase * 384
            o23 = phase * 320
            oo = phase * OUTP

            def bcast_i(s):
                return jnp.broadcast_to(s, (16,)).astype(jnp.int32)

            def bcast_f(s):
                return jnp.broadcast_to(s, (16,)).astype(jnp.float32)

            def do_level(dref, dbase, ybase, w, inv_s, obase):
                xs = c0 * inv_s
                ys = c1 * inv_s
                xi = floori(xs)
                fx = xs - xi.astype(jnp.float32)
                yi = floori(ys)
                fy = ys - yi.astype(jnp.float32)
                x0 = xi - 4
                y0 = yi - 4
                one_v = jnp.full((16,), 1.0, jnp.float32)
                zero_v = jnp.zeros((16,), jnp.float32)
                xa = bcast_i(x0) + iota
                xb = xa + 1
                va = jnp.where((xa >= 0) & (xa <= w - 1), one_v, zero_v)
                vb = jnp.where((xb >= 0) & (xb <= w - 1), one_v, zero_v)
                ia = jnp.clip(xa, 0, w - 1)
                ib = jnp.clip(xb, 0, w - 1)
                gav = bcast_f(1.0 - fx) * va
                gbv = bcast_f(fx) * vb
                hrows = []
                for jj in range(10):
                    l = jnp.clip(y0 + jj, 0, w - 1) - ybase
                    offv = bcast_i(dbase + l * w)
                    ra = plsc.load_gather(dref, [offv + ia])
                    rb = plsc.load_gather(dref, [offv + ib])
                    hrows.append(gav * ra + gbv * rb)
                oidx = oo + obase + iota * NT
                for k in range(NT):
                    ya = y0 + k
                    yb = ya + 1
                    wa = jnp.where((ya >= 0) & (ya <= w - 1), 1.0 - fy, 0.0)
                    wb = jnp.where((yb >= 0) & (yb <= w - 1), fy, 0.0)
                    outk = bcast_f(wa) * hrows[k] + bcast_f(wb) * hrows[k + 1]
                    plsc.store_scatter(obuf, [oidx + k], outk, mask=lane9)

            do_level(b0, o0, ts0, 64, 1.0, 0)
            do_level(b1, o1, 2 * ts1, 32, 0.5, 81)
            do_level(b23, o23, 0, 16, 0.25, 162)
            do_level(b23, o23 + 256, 0, 8, 0.125, 243)

            wait_out()
            pltpu.make_async_copy(
                obuf.at[pl.ds(oo, OUTP)],
                out_hbm.at[pl.ds((q0 + qi) * OUTP, OUTP)], osem).start()
            return carry

        lax.fori_loop(0, QPT, body, 0)
        wait_out()
        wait_in(0, sa0, sa1, sa2)

    return ker(pyr_flat, cq)


def kernel(fmap1, fmap2, coords):
    B, C, H, W = fmap1.shape
    f1t = fmap1.reshape(C, H * W).T
    f2m = fmap2.reshape(C, H * W)
    g1 = _pool_mat(64)
    g2 = _pool_mat(32)
    g3 = _pool_mat(16)
    pyr = _tc_pyramid(f1t, f2m, g1, g2, g3)
    cq = coords.reshape(2, H * W).T.reshape(-1)
    out = _sc_sample(pyr.reshape(-1), cq)
    out = out.reshape(NQ, OUTP)[:, :OUTC]
    return out.reshape(1, OUTC, H, W).astype(jnp.float32)
